# baseline (speedup 1.0000x reference)
import functools

import jax
import jax.numpy as jnp
from jax.experimental import pallas as pl
from jax.experimental.pallas import tpu as pltpu


def _round_up(n, m):
    return ((n + m - 1) // m) * m


def _head_kernel(x_ref, w_ref, b_ref, o_ref, *, inv_nm1):
    tok_sum = jnp.sum(x_ref[...], axis=1, dtype=jnp.float32)
    avg = (tok_sum - x_ref[:, 0, :]) * inv_nm1
    out = jnp.dot(avg, w_ref[...], preferred_element_type=jnp.float32)
    o_ref[...] = out + b_ref[...]


def kernel(x, w, b):
    B, S, D = x.shape
    D_in, C = w.shape

    C_pad = _round_up(C, 128)
    if C_pad != C:
        w = jnp.pad(w, ((0, 0), (0, C_pad - C)))
        b = jnp.pad(b, (0, C_pad - C))
    b2 = b.reshape(1, C_pad)

    itemsize = x.dtype.itemsize
    TILE_B = 32
    while TILE_B > 1 and B % TILE_B != 0:
        TILE_B //= 2
    nb = B // TILE_B

    cost = pl.CostEstimate(
        flops=2 * B * D_in * C_pad + B * S * D,
        transcendentals=0,
        bytes_accessed=(B * S * D * itemsize
                        + D_in * C_pad * w.dtype.itemsize
                        + B * C_pad * 4),
    )
    out = pl.pallas_call(
        functools.partial(_head_kernel, inv_nm1=1.0 / (S - 1)),
        out_shape=jax.ShapeDtypeStruct((B, C_pad), jnp.float32),
        grid=(nb,),
        in_specs=[
            pl.BlockSpec((TILE_B, S, D), lambda i: (i, 0, 0)),
            pl.BlockSpec((D_in, C_pad), lambda i: (0, 0)),
            pl.BlockSpec((1, C_pad), lambda i: (0, 0)),
        ],
        out_specs=pl.BlockSpec((TILE_B, C_pad), lambda i: (i, 0)),
        compiler_params=pltpu.CompilerParams(
            dimension_semantics=("parallel",),
            vmem_limit_bytes=56 * 1024 * 1024,
        ),
        cost_estimate=cost,
    )(x, w, b2)

    return out[:, :C] if C_pad != C else out


# speedup vs baseline: 1.0188x; 1.0188x over previous
import functools

import jax
import jax.numpy as jnp
from jax.experimental import pallas as pl
from jax.experimental.pallas import tpu as pltpu


def _round_up(n, m):
    return ((n + m - 1) // m) * m


def _head_kernel(x_ref, w_ref, b_ref, o_ref, *, inv_nm1):
    tok_sum = jnp.sum(x_ref[...], axis=1, dtype=jnp.float32)
    avg = (tok_sum - x_ref[:, 0, :]) * inv_nm1
    out = jnp.dot(avg, w_ref[...], preferred_element_type=jnp.float32)
    o_ref[...] = out + b_ref[...]


def kernel(x, w, b):
    B, S, D = x.shape
    D_in, C = w.shape

    C_pad = _round_up(C, 128)
    if C_pad != C:
        w = jnp.pad(w, ((0, 0), (0, C_pad - C)))
        b = jnp.pad(b, (0, C_pad - C))
    b2 = b.reshape(1, C_pad)

    itemsize = x.dtype.itemsize
    TILE_B = 16
    while TILE_B > 1 and B % TILE_B != 0:
        TILE_B //= 2
    nb = B // TILE_B

    cost = pl.CostEstimate(
        flops=2 * B * D_in * C_pad + B * S * D,
        transcendentals=0,
        bytes_accessed=(B * S * D * itemsize
                        + D_in * C_pad * w.dtype.itemsize
                        + B * C_pad * 4),
    )
    out = pl.pallas_call(
        functools.partial(_head_kernel, inv_nm1=1.0 / (S - 1)),
        out_shape=jax.ShapeDtypeStruct((B, C_pad), jnp.float32),
        grid=(nb,),
        in_specs=[
            pl.BlockSpec((TILE_B, S, D), lambda i: (i, 0, 0)),
            pl.BlockSpec((D_in, C_pad), lambda i: (0, 0)),
            pl.BlockSpec((1, C_pad), lambda i: (0, 0)),
        ],
        out_specs=pl.BlockSpec((TILE_B, C_pad), lambda i: (i, 0)),
        compiler_params=pltpu.CompilerParams(
            dimension_semantics=("parallel",),
            vmem_limit_bytes=56 * 1024 * 1024,
        ),
        cost_estimate=cost,
    )(x, w, b2)

    return out[:, :C] if C_pad != C else out


# speedup vs baseline: 1.0462x; 1.0270x over previous
import functools

import jax
import jax.numpy as jnp
from jax.experimental import pallas as pl
from jax.experimental.pallas import tpu as pltpu


def _round_up(n, m):
    return ((n + m - 1) // m) * m


def _head_kernel(x_ref, w_ref, b_ref, o_ref, *, inv_nm1):
    s = pl.program_id(1)
    tok_sum = jnp.sum(x_ref[...], axis=1, dtype=jnp.float32)

    @pl.when(s == 0)
    def _first():
        part = (tok_sum - x_ref[:, 0, :]) * inv_nm1
        o_ref[...] = (jnp.dot(part, w_ref[...],
                              preferred_element_type=jnp.float32)
                      + b_ref[...])

    @pl.when(s > 0)
    def _rest():
        part = tok_sum * inv_nm1
        o_ref[...] += jnp.dot(part, w_ref[...],
                              preferred_element_type=jnp.float32)


def kernel(x, w, b):
    B, S, D = x.shape
    D_in, C = w.shape

    C_pad = _round_up(C, 128)
    if C_pad != C:
        w = jnp.pad(w, ((0, 0), (0, C_pad - C)))
        b = jnp.pad(b, (0, C_pad - C))
    b2 = b.reshape(1, C_pad)

    TILE_B = 16
    while TILE_B > 1 and B % TILE_B != 0:
        TILE_B //= 2
    nb = B // TILE_B
    ns = 2 if S % 2 == 0 and S >= 16 else 1
    TILE_S = S // ns

    itemsize = x.dtype.itemsize
    cost = pl.CostEstimate(
        flops=2 * B * D_in * C_pad * ns + B * S * D,
        transcendentals=0,
        bytes_accessed=(B * S * D * itemsize
                        + D_in * C_pad * w.dtype.itemsize
                        + B * C_pad * 4),
    )
    out = pl.pallas_call(
        functools.partial(_head_kernel, inv_nm1=1.0 / (S - 1)),
        out_shape=jax.ShapeDtypeStruct((B, C_pad), jnp.float32),
        grid=(nb, ns),
        in_specs=[
            pl.BlockSpec((TILE_B, TILE_S, D), lambda i, s: (i, s, 0)),
            pl.BlockSpec((D_in, C_pad), lambda i, s: (0, 0)),
            pl.BlockSpec((1, C_pad), lambda i, s: (0, 0)),
        ],
        out_specs=pl.BlockSpec((TILE_B, C_pad), lambda i, s: (i, 0)),
        compiler_params=pltpu.CompilerParams(
            dimension_semantics=("parallel", "arbitrary"),
            vmem_limit_bytes=48 * 1024 * 1024,
        ),
        cost_estimate=cost,
    )(x, w, b2)

    return out[:, :C] if C_pad != C else out


# speedup vs baseline: 1.0607x; 1.0139x over previous
import functools

import jax
import jax.numpy as jnp
from jax.experimental import pallas as pl
from jax.experimental.pallas import tpu as pltpu


def _round_up(n, m):
    return ((n + m - 1) // m) * m


def _head_kernel(x_ref, w_ref, b_ref, o_ref, *, inv_nm1):
    s = pl.program_id(1)
    tok_sum = jnp.sum(x_ref[...], axis=1, dtype=jnp.float32)

    @pl.when(s == 0)
    def _first():
        part = (tok_sum - x_ref[:, 0, :]) * inv_nm1
        o_ref[...] = (jnp.dot(part, w_ref[...],
                              preferred_element_type=jnp.float32)
                      + b_ref[...])

    @pl.when(s > 0)
    def _rest():
        part = tok_sum * inv_nm1
        o_ref[...] += jnp.dot(part, w_ref[...],
                              preferred_element_type=jnp.float32)


def kernel(x, w, b):
    B, S, D = x.shape
    D_in, C = w.shape

    C_pad = _round_up(C, 128)
    if C_pad != C:
        w = jnp.pad(w, ((0, 0), (0, C_pad - C)))
        b = jnp.pad(b, (0, C_pad - C))
    b2 = b.reshape(1, C_pad)

    TILE_B = 16
    while TILE_B > 1 and B % TILE_B != 0:
        TILE_B //= 2
    nb = B // TILE_B
    ns = 1
    TILE_S = S // ns

    itemsize = x.dtype.itemsize
    cost = pl.CostEstimate(
        flops=2 * B * D_in * C_pad * ns + B * S * D,
        transcendentals=0,
        bytes_accessed=(B * S * D * itemsize
                        + D_in * C_pad * w.dtype.itemsize
                        + B * C_pad * 4),
    )
    out = pl.pallas_call(
        functools.partial(_head_kernel, inv_nm1=1.0 / (S - 1)),
        out_shape=jax.ShapeDtypeStruct((B, C_pad), jnp.float32),
        grid=(nb, ns),
        in_specs=[
            pl.BlockSpec((TILE_B, TILE_S, D), lambda i, s: (i, s, 0)),
            pl.BlockSpec((D_in, C_pad), lambda i, s: (0, 0)),
            pl.BlockSpec((1, C_pad), lambda i, s: (0, 0)),
        ],
        out_specs=pl.BlockSpec((TILE_B, C_pad), lambda i, s: (i, 0)),
        compiler_params=pltpu.CompilerParams(
            dimension_semantics=("parallel", "arbitrary"),
            vmem_limit_bytes=48 * 1024 * 1024,
        ),
        cost_estimate=cost,
    )(x, w, b2)

    return out[:, :C] if C_pad != C else out
